# Initial kernel scaffold; baseline (speedup 1.0000x reference)
import jax
import jax.numpy as jnp
from jax.experimental import pallas as pl


def kernel(x, edge_index, params):
    raise NotImplementedError("write your pallas kernel here")


# speedup vs baseline: 13.7223x; 13.7223x over previous
import jax
import jax.numpy as jnp
from jax import lax
from jax.experimental import pallas as pl
from jax.experimental.pallas import tpu as pltpu
from jax.experimental.pallas import tpu_sc as plsc

_N = 10000
_E = 320000
_DIN = 128
_H = 64
_NC = 2
_NS = 16
_NW = _NC * _NS
_C = 128
_NCH = 80
_EPW = _C * _NCH
_EPAD = _NW * _EPW
_NACC = 10240
_RPT = _NACC // _NS
_OPT = _N // _NS


_NBUF = 8


def _sc_body(g_hbm, srci_hbm, dsti_hbm, z_hbm, out_hbm,
             src_v, dst_v, rows, acc, sem_g, sem_s):
    cid = lax.axis_index("c")
    sid = lax.axis_index("s")
    wid = cid * _NS + sid
    ob = sid * _RPT
    pltpu.sync_copy(z_hbm, rows[0])
    for j in range(_RPT // _C):
        pltpu.sync_copy(rows[0], acc.at[pl.ds(ob + j * _C, _C)])
    pltpu.sync_copy(srci_hbm.at[wid], src_v)
    pltpu.sync_copy(dsti_hbm.at[wid], dst_v)
    plsc.subcore_barrier()

    for b in range(_NBUF):
        pltpu.async_copy(g_hbm.at[src_v.at[b]], rows[b], sem_g[b])

    def ring(k4, carry):
        for b in range(_NBUF):
            k = _NBUF * k4 + b
            pltpu.make_async_copy(g_hbm.at[src_v.at[k]], rows[b],
                                  sem_g[b]).wait()
            pltpu.async_copy(rows[b], acc.at[dst_v.at[k]], sem_s[b], add=True)
        for b in range(_NBUF):
            k = _NBUF * k4 + b
            pltpu.make_async_copy(rows[b], acc.at[dst_v.at[k]],
                                  sem_s[b]).wait()
            kn = jnp.minimum(k + _NBUF, _NCH - 1)
            pltpu.async_copy(g_hbm.at[src_v.at[kn]], rows[b], sem_g[b])
        return carry

    lax.fori_loop(0, _NCH // _NBUF, ring, 0)
    for b in range(_NBUF):
        pltpu.make_async_copy(g_hbm.at[src_v.at[_NCH - 1]], rows[b],
                              sem_g[b]).wait()
    plsc.subcore_barrier()
    for j in range(_RPT // _C):
        pltpu.sync_copy(acc.at[pl.ds(ob + j * _C, _C)], rows[0])
        pltpu.sync_copy(rows[0], out_hbm.at[pl.ds(cid * _NACC + ob + j * _C, _C)])


def _sc_scatter(g, srcp, dstp, zeros):
    call = pl.kernel(
        _sc_body,
        out_type=jax.ShapeDtypeStruct((_NC * _NACC, _H), jnp.float32),
        mesh=plsc.VectorSubcoreMesh(core_axis_name="c", subcore_axis_name="s",
                                    num_cores=_NC, num_subcores=_NS),
        compiler_params=pltpu.CompilerParams(use_tc_tiling_on_sc=False),
        scratch_types=[
            pltpu.VMEM((_NCH, _C), jnp.int32),
            pltpu.VMEM((_NCH, _C), jnp.int32),
            [pltpu.VMEM((_C, _H), jnp.float32) for _ in range(_NBUF)],
            pltpu.VMEM_SHARED((_NACC, _H), jnp.float32),
            [pltpu.SemaphoreType.DMA for _ in range(_NBUF)],
            [pltpu.SemaphoreType.DMA for _ in range(_NBUF)],
        ],
    )
    return call(g, srcp, dstp, zeros)


def _tc0_body(x_ref, w1_ref, wp_ref, bp_ref, g_ref, sc_ref):
    x = x_ref[...]
    g_ref[0:_N] = jnp.dot(x, w1_ref[...], preferred_element_type=jnp.float32)
    g_ref[_N:_NACC] = jnp.zeros((_NACC - _N, _H), jnp.float32)
    cs = jnp.sum(x, axis=0, keepdims=True)
    sc_ref[...] = jnp.dot(cs, wp_ref[...],
                          preferred_element_type=jnp.float32) + bp_ref[...]


def _mlp_body(g_ref, p_ref, eps_ref, b1_ref, g1_ref, bb1_ref, w2_ref, b2_ref,
              go_ref, bo_ref, wn_ref, wp_ref, bp_ref, gn_ref, sc_ref):
    z1 = ((1.0 + eps_ref[0, 0]) * g_ref[0:_N]
          + p_ref[0:_N] + p_ref[_NACC:_NACC + _N] + b1_ref[...])
    m1 = jnp.mean(z1, axis=0, keepdims=True)
    v1 = jnp.mean((z1 - m1) ** 2, axis=0, keepdims=True)
    y = jnp.maximum(g1_ref[...] * (z1 - m1) * lax.rsqrt(v1 + 1e-5)
                    + bb1_ref[...], 0.0)
    z2 = jnp.dot(y, w2_ref[...], preferred_element_type=jnp.float32) + b2_ref[...]
    m2 = jnp.mean(z2, axis=0, keepdims=True)
    v2 = jnp.mean((z2 - m2) ** 2, axis=0, keepdims=True)
    h = jnp.maximum(go_ref[...] * (z2 - m2) * lax.rsqrt(v2 + 1e-5)
                    + bo_ref[...], 0.0)
    gn_ref[0:_N] = jnp.dot(h, wn_ref[...], preferred_element_type=jnp.float32)
    gn_ref[_N:_NACC] = jnp.zeros((_NACC - _N, gn_ref.shape[1]), jnp.float32)
    cs = jnp.sum(h, axis=0, keepdims=True)
    sc_ref[...] = jnp.dot(cs, wp_ref[...],
                          preferred_element_type=jnp.float32) + bp_ref[...]


def _tc0(x, w1, wp, bp):
    return pl.pallas_call(
        _tc0_body,
        out_shape=[jax.ShapeDtypeStruct((_NACC, _H), jnp.float32),
                   jax.ShapeDtypeStruct((1, 1), jnp.float32)],
    )(x, w1, wp, bp.reshape(1, 1))


def _mlp(g, parts, prm, l):
    wn = prm[f"W1_{l + 1}"] if l < 3 else prm["Wp_4"]
    gn_d = _H if l < 3 else 1
    return pl.pallas_call(
        _mlp_body,
        out_shape=[jax.ShapeDtypeStruct((_NACC, gn_d), jnp.float32),
                   jax.ShapeDtypeStruct((1, 1), jnp.float32)],
    )(g, parts, prm[f"eps_{l}"].reshape(1, 1),
      prm[f"b1_{l}"].reshape(1, _H), prm[f"g1_{l}"].reshape(1, _H),
      prm[f"bb1_{l}"].reshape(1, _H), prm[f"W2_{l}"],
      prm[f"b2_{l}"].reshape(1, _H), prm[f"go_{l}"].reshape(1, _H),
      prm[f"bo_{l}"].reshape(1, _H), wn, prm[f"Wp_{l + 1}"],
      prm[f"bp_{l + 1}"].reshape(1, 1))


def kernel(x, edge_index, params):
    src = edge_index[0]
    dst = edge_index[1]
    npad = _EPAD - _E
    ar = jnp.arange(npad, dtype=jnp.int32)
    srcp = jnp.concatenate([src, ar % _N]).reshape(_NW, _NCH, _C)
    dstp = jnp.concatenate([dst, _N + ar % (_NACC - _N)]).reshape(_NW, _NCH, _C)
    zeros = jnp.zeros((_C, _H), jnp.float32)

    g, out = _tc0(x, params["W1_0"], params["Wp_0"], params["bp_0"])
    for l in range(4):
        parts = _sc_scatter(g, srcp, dstp, zeros)
        g, s = _mlp(g, parts, params, l)
        out = out + s
    return out


# speedup vs baseline: 17.1984x; 1.2533x over previous
import jax
import jax.numpy as jnp
from jax import lax
from jax.experimental import pallas as pl
from jax.experimental.pallas import tpu as pltpu
from jax.experimental.pallas import tpu_sc as plsc

_N = 10000
_E = 320000
_DIN = 128
_H = 64
_NC = 2
_NS = 16
_NW = _NC * _NS
_C = 128
_NCH = 80
_EPW = _C * _NCH
_EPAD = _NW * _EPW
_NACC = 10240
_RPT = _NACC // _NS
_OPT = _N // _NS


_NBUF = 8


def _sc_body(g_hbm, srci_hbm, dsti_hbm, z_hbm, out_hbm,
             src_v, dst_v, rows, acc, sem_g, sem_s):
    cid = lax.axis_index("c")
    sid = lax.axis_index("s")
    wid = cid * _NS + sid
    ob = sid * _RPT
    pltpu.sync_copy(z_hbm, rows[0])
    for j in range(_RPT // _C):
        pltpu.sync_copy(rows[0], acc.at[pl.ds(ob + j * _C, _C)])
    pltpu.sync_copy(srci_hbm.at[wid], src_v)
    pltpu.sync_copy(dsti_hbm.at[wid], dst_v)
    plsc.subcore_barrier()

    for b in range(_NBUF):
        pltpu.async_copy(g_hbm.at[src_v.at[b]], rows[b], sem_g[b])

    def ring(k4, carry):
        for b in range(_NBUF):
            k = _NBUF * k4 + b
            pltpu.make_async_copy(g_hbm.at[src_v.at[k]], rows[b],
                                  sem_g[b]).wait()
            pltpu.async_copy(rows[b], acc.at[dst_v.at[k]], sem_s[b], add=True)
        for b in range(_NBUF):
            k = _NBUF * k4 + b
            pltpu.make_async_copy(rows[b], acc.at[dst_v.at[k]],
                                  sem_s[b]).wait()
            kn = jnp.minimum(k + _NBUF, _NCH - 1)
            pltpu.async_copy(g_hbm.at[src_v.at[kn]], rows[b], sem_g[b])
        return carry

    lax.fori_loop(0, _NCH // _NBUF, ring, 0)
    for b in range(_NBUF):
        pltpu.make_async_copy(g_hbm.at[src_v.at[_NCH - 1]], rows[b],
                              sem_g[b]).wait()
    plsc.subcore_barrier()
    for j in range(_RPT // _C):
        pltpu.sync_copy(acc.at[pl.ds(ob + j * _C, _C)], rows[0])
        pltpu.sync_copy(rows[0],
                        out_hbm.at[pl.ds(cid * _NACC + ob + j * _C, _C)])


def _sc_scatter(g, srcp, dstp, zeros):
    call = pl.kernel(
        _sc_body,
        out_type=jax.ShapeDtypeStruct((_NC * _NACC, _H), jnp.bfloat16),
        mesh=plsc.VectorSubcoreMesh(core_axis_name="c", subcore_axis_name="s",
                                    num_cores=_NC, num_subcores=_NS),
        compiler_params=pltpu.CompilerParams(use_tc_tiling_on_sc=False),
        scratch_types=[
            pltpu.VMEM((_NCH, _C), jnp.int32),
            pltpu.VMEM((_NCH, _C), jnp.int32),
            [pltpu.VMEM((_C, _H), jnp.bfloat16) for _ in range(_NBUF)],
            pltpu.VMEM_SHARED((_NACC, _H), jnp.bfloat16),
            [pltpu.SemaphoreType.DMA for _ in range(_NBUF)],
            [pltpu.SemaphoreType.DMA for _ in range(_NBUF)],
        ],
    )
    return call(g, srcp, dstp, zeros)


def _tc0_body(x_ref, w1_ref, wp_ref, bp_ref, g_ref, gb_ref, sc_ref):
    x = x_ref[...]
    g = jnp.dot(x, w1_ref[...], preferred_element_type=jnp.float32)
    g_ref[0:_N] = g
    g_ref[_N:_NACC] = jnp.zeros((_NACC - _N, _H), jnp.float32)
    gb_ref[0:_N] = g.astype(jnp.bfloat16)
    gb_ref[_N:_NACC] = jnp.zeros((_NACC - _N, _H), jnp.bfloat16)
    cs = jnp.sum(x, axis=0, keepdims=True)
    sc_ref[...] = jnp.dot(cs, wp_ref[...],
                          preferred_element_type=jnp.float32) + bp_ref[...]


def _mlp_body(g_ref, p_ref, eps_ref, b1_ref, g1_ref, bb1_ref, w2_ref, b2_ref,
              go_ref, bo_ref, wn_ref, wp_ref, bp_ref, gn_ref, gnb_ref, sc_ref):
    agg = (p_ref[0:_N].astype(jnp.float32)
           + p_ref[_NACC:_NACC + _N].astype(jnp.float32))
    z1 = (1.0 + eps_ref[0, 0]) * g_ref[0:_N] + agg + b1_ref[...]
    m1 = jnp.mean(z1, axis=0, keepdims=True)
    v1 = jnp.mean((z1 - m1) ** 2, axis=0, keepdims=True)
    y = jnp.maximum(g1_ref[...] * (z1 - m1) * lax.rsqrt(v1 + 1e-5)
                    + bb1_ref[...], 0.0)
    z2 = jnp.dot(y, w2_ref[...], preferred_element_type=jnp.float32) + b2_ref[...]
    m2 = jnp.mean(z2, axis=0, keepdims=True)
    v2 = jnp.mean((z2 - m2) ** 2, axis=0, keepdims=True)
    h = jnp.maximum(go_ref[...] * (z2 - m2) * lax.rsqrt(v2 + 1e-5)
                    + bo_ref[...], 0.0)
    gn = jnp.dot(h, wn_ref[...], preferred_element_type=jnp.float32)
    gn_ref[0:_N] = gn
    gn_ref[_N:_NACC] = jnp.zeros((_NACC - _N, gn_ref.shape[1]), jnp.float32)
    gnb_ref[0:_N] = gn.astype(jnp.bfloat16)
    gnb_ref[_N:_NACC] = jnp.zeros((_NACC - _N, gn_ref.shape[1]), jnp.bfloat16)
    cs = jnp.sum(h, axis=0, keepdims=True)
    sc_ref[...] = jnp.dot(cs, wp_ref[...],
                          preferred_element_type=jnp.float32) + bp_ref[...]


def _tc0(x, w1, wp, bp):
    return pl.pallas_call(
        _tc0_body,
        out_shape=[jax.ShapeDtypeStruct((_NACC, _H), jnp.float32),
                   jax.ShapeDtypeStruct((_NACC, _H), jnp.bfloat16),
                   jax.ShapeDtypeStruct((1, 1), jnp.float32)],
    )(x, w1, wp, bp.reshape(1, 1))


def _mlp(g, parts, prm, l):
    wn = prm[f"W1_{l + 1}"] if l < 3 else prm["Wp_4"]
    gn_d = _H if l < 3 else 1
    return pl.pallas_call(
        _mlp_body,
        out_shape=[jax.ShapeDtypeStruct((_NACC, gn_d), jnp.float32),
                   jax.ShapeDtypeStruct((_NACC, gn_d), jnp.bfloat16),
                   jax.ShapeDtypeStruct((1, 1), jnp.float32)],
    )(g, parts, prm[f"eps_{l}"].reshape(1, 1),
      prm[f"b1_{l}"].reshape(1, _H), prm[f"g1_{l}"].reshape(1, _H),
      prm[f"bb1_{l}"].reshape(1, _H), prm[f"W2_{l}"],
      prm[f"b2_{l}"].reshape(1, _H), prm[f"go_{l}"].reshape(1, _H),
      prm[f"bo_{l}"].reshape(1, _H), wn, prm[f"Wp_{l + 1}"],
      prm[f"bp_{l + 1}"].reshape(1, 1))


def kernel(x, edge_index, params):
    src = edge_index[0]
    dst = edge_index[1]
    npad = _EPAD - _E
    ar = jnp.arange(npad, dtype=jnp.int32)
    srcp = jnp.concatenate([src, ar % _N]).reshape(_NW, _NCH, _C)
    dstp = jnp.concatenate([dst, _N + ar % (_NACC - _N)]).reshape(_NW, _NCH, _C)
    zeros = jnp.zeros((_C, _H), jnp.bfloat16)

    g, gb, out = _tc0(x, params["W1_0"], params["Wp_0"], params["bp_0"])
    for l in range(4):
        parts = _sc_scatter(gb, srcp, dstp, zeros)
        g, gb, s = _mlp(g, parts, params, l)
        out = out + s
    return out


# speedup vs baseline: 17.4267x; 1.0133x over previous
import jax
import jax.numpy as jnp
from jax import lax
from jax.experimental import pallas as pl
from jax.experimental.pallas import tpu as pltpu
from jax.experimental.pallas import tpu_sc as plsc

_N = 10000
_E = 320000
_DIN = 128
_H = 64
_NC = 2
_NS = 16
_NW = _NC * _NS
_C = 125
_NCH = 80
_CZ = 128
_NACC = 10240
_RPT = _NACC // _NS


_NBUF = 8


def _sc_body(g_hbm, srci_hbm, dsti_hbm, z_hbm, out_hbm,
             src_v, dst_v, rows, acc, sem_g, sem_s):
    cid = lax.axis_index("c")
    sid = lax.axis_index("s")
    wid = cid * _NS + sid
    ob = sid * _RPT
    pltpu.sync_copy(z_hbm, rows[0])
    for j in range(_RPT // _CZ):
        pltpu.sync_copy(rows[0], acc.at[pl.ds(ob + j * _CZ, _CZ)])
    pltpu.sync_copy(srci_hbm.at[wid], src_v)
    pltpu.sync_copy(dsti_hbm.at[wid], dst_v)
    plsc.subcore_barrier()

    for b in range(_NBUF):
        pltpu.async_copy(g_hbm.at[src_v.at[b]], rows[b].at[pl.ds(0, _C)],
                         sem_g[b])

    def ring(k4, carry):
        for b in range(_NBUF):
            k = _NBUF * k4 + b
            pltpu.make_async_copy(g_hbm.at[src_v.at[k]],
                                  rows[b].at[pl.ds(0, _C)], sem_g[b]).wait()
            pltpu.async_copy(rows[b].at[pl.ds(0, _C)], acc.at[dst_v.at[k]],
                             sem_s[b], add=True)
        for b in range(_NBUF):
            k = _NBUF * k4 + b
            pltpu.make_async_copy(rows[b].at[pl.ds(0, _C)],
                                  acc.at[dst_v.at[k]], sem_s[b]).wait()
            kn = jnp.minimum(k + _NBUF, _NCH - 1)
            pltpu.async_copy(g_hbm.at[src_v.at[kn]],
                             rows[b].at[pl.ds(0, _C)], sem_g[b])
        return carry

    lax.fori_loop(0, _NCH // _NBUF, ring, 0)
    for b in range(_NBUF):
        pltpu.make_async_copy(g_hbm.at[src_v.at[_NCH - 1]],
                              rows[b].at[pl.ds(0, _C)], sem_g[b]).wait()
    plsc.subcore_barrier()
    for j in range(_RPT // _CZ):
        pltpu.sync_copy(acc.at[pl.ds(ob + j * _CZ, _CZ)], rows[0])
        pltpu.sync_copy(rows[0],
                        out_hbm.at[pl.ds(cid * _NACC + ob + j * _CZ, _CZ)])


def _sc_scatter(g, srcp, dstp, zeros):
    call = pl.kernel(
        _sc_body,
        out_type=jax.ShapeDtypeStruct((_NC * _NACC, _H), jnp.bfloat16),
        mesh=plsc.VectorSubcoreMesh(core_axis_name="c", subcore_axis_name="s",
                                    num_cores=_NC, num_subcores=_NS),
        compiler_params=pltpu.CompilerParams(use_tc_tiling_on_sc=False),
        scratch_types=[
            pltpu.VMEM((_NCH, _C), jnp.int32),
            pltpu.VMEM((_NCH, _C), jnp.int32),
            [pltpu.VMEM((_CZ, _H), jnp.bfloat16) for _ in range(_NBUF)],
            pltpu.VMEM_SHARED((_NACC, _H), jnp.bfloat16),
            [pltpu.SemaphoreType.DMA for _ in range(_NBUF)],
            [pltpu.SemaphoreType.DMA for _ in range(_NBUF)],
        ],
    )
    return call(g, srcp, dstp, zeros)


def _tc0_body(x_ref, w1_ref, wp_ref, bp_ref, g_ref, gb_ref, sc_ref):
    x = x_ref[...]
    g = jnp.dot(x, w1_ref[...], preferred_element_type=jnp.float32)
    g_ref[0:_N] = g
    g_ref[_N:_NACC] = jnp.zeros((_NACC - _N, _H), jnp.float32)
    gb_ref[0:_N] = g.astype(jnp.bfloat16)
    gb_ref[_N:_NACC] = jnp.zeros((_NACC - _N, _H), jnp.bfloat16)
    cs = jnp.sum(x, axis=0, keepdims=True)
    sc_ref[...] = jnp.dot(cs, wp_ref[...],
                          preferred_element_type=jnp.float32) + bp_ref[...]


def _mlp_body(g_ref, p_ref, eps_ref, b1_ref, g1_ref, bb1_ref, w2_ref, b2_ref,
              go_ref, bo_ref, wn_ref, wp_ref, bp_ref, gn_ref, gnb_ref, sc_ref):
    agg = (p_ref[0:_N].astype(jnp.float32)
           + p_ref[_NACC:_NACC + _N].astype(jnp.float32))
    z1 = (1.0 + eps_ref[0, 0]) * g_ref[0:_N] + agg + b1_ref[...]
    m1 = jnp.mean(z1, axis=0, keepdims=True)
    v1 = jnp.mean((z1 - m1) ** 2, axis=0, keepdims=True)
    y = jnp.maximum(g1_ref[...] * (z1 - m1) * lax.rsqrt(v1 + 1e-5)
                    + bb1_ref[...], 0.0)
    z2 = jnp.dot(y, w2_ref[...], preferred_element_type=jnp.float32) + b2_ref[...]
    m2 = jnp.mean(z2, axis=0, keepdims=True)
    v2 = jnp.mean((z2 - m2) ** 2, axis=0, keepdims=True)
    h = jnp.maximum(go_ref[...] * (z2 - m2) * lax.rsqrt(v2 + 1e-5)
                    + bo_ref[...], 0.0)
    gn = jnp.dot(h, wn_ref[...], preferred_element_type=jnp.float32)
    gn_ref[0:_N] = gn
    gn_ref[_N:_NACC] = jnp.zeros((_NACC - _N, gn_ref.shape[1]), jnp.float32)
    gnb_ref[0:_N] = gn.astype(jnp.bfloat16)
    gnb_ref[_N:_NACC] = jnp.zeros((_NACC - _N, gn_ref.shape[1]), jnp.bfloat16)
    cs = jnp.sum(h, axis=0, keepdims=True)
    sc_ref[...] = jnp.dot(cs, wp_ref[...],
                          preferred_element_type=jnp.float32) + bp_ref[...]


def _tc0(x, w1, wp, bp):
    return pl.pallas_call(
        _tc0_body,
        out_shape=[jax.ShapeDtypeStruct((_NACC, _H), jnp.float32),
                   jax.ShapeDtypeStruct((_NACC, _H), jnp.bfloat16),
                   jax.ShapeDtypeStruct((1, 1), jnp.float32)],
    )(x, w1, wp, bp.reshape(1, 1))


def _mlp(g, parts, prm, l):
    wn = prm[f"W1_{l + 1}"] if l < 3 else prm["Wp_4"]
    gn_d = _H if l < 3 else 1
    return pl.pallas_call(
        _mlp_body,
        out_shape=[jax.ShapeDtypeStruct((_NACC, gn_d), jnp.float32),
                   jax.ShapeDtypeStruct((_NACC, gn_d), jnp.bfloat16),
                   jax.ShapeDtypeStruct((1, 1), jnp.float32)],
    )(g, parts, prm[f"eps_{l}"].reshape(1, 1),
      prm[f"b1_{l}"].reshape(1, _H), prm[f"g1_{l}"].reshape(1, _H),
      prm[f"bb1_{l}"].reshape(1, _H), prm[f"W2_{l}"],
      prm[f"b2_{l}"].reshape(1, _H), prm[f"go_{l}"].reshape(1, _H),
      prm[f"bo_{l}"].reshape(1, _H), wn, prm[f"Wp_{l + 1}"],
      prm[f"bp_{l + 1}"].reshape(1, 1))


def kernel(x, edge_index, params):
    srcp = edge_index[0].reshape(_NW, _NCH, _C)
    dstp = edge_index[1].reshape(_NW, _NCH, _C)
    zeros = jnp.zeros((_CZ, _H), jnp.bfloat16)

    g, gb, out = _tc0(x, params["W1_0"], params["Wp_0"], params["bp_0"])
    for l in range(4):
        parts = _sc_scatter(gb, srcp, dstp, zeros)
        g, gb, s = _mlp(g, parts, params, l)
        out = out + s
    return out
